# baseline (speedup 1.0000x reference)
import functools

import jax
import jax.numpy as jnp
import numpy as np
from jax.experimental import pallas as pl
from jax.experimental.pallas import tpu as pltpu

_LANES = 128


def _roll_lanes(v, k):
    if k == 0:
        return v
    return jnp.concatenate([v[:, k:], v[:, :k]], axis=1)


def _conv_pool_kernel(x_ref, w1_ref, b_ref, hwm_ref, dm_ref, sc_ref,
                      pooled_ref, psum_ref, psq_ref, *, dims):
    g, cin, cout, h, w = dims
    hw = h * w
    bf16 = jnp.bfloat16

    mrows = []
    for e in range(g):
        x = x_ref[e].astype(bf16)

        slabs = []
        for kh in range(3):
            for kw in range(3):
                j = kh * 3 + kw
                sh = (kh - 1) * w + (kw - 1)
                sl = _roll_lanes(x, sh)
                if j != 4:
                    sl = sl * hwm_ref[j:j + 1, :]
                slabs.append(sl)
        u = jnp.concatenate(slabs, axis=0)

        t = jnp.dot(w1_ref[...], u,
                    preferred_element_type=jnp.float32)

        y = (_roll_lanes(t[:cout], -hw) * dm_ref[0:1, :]
             + t[cout:2 * cout]
             + _roll_lanes(t[2 * cout:], hw) * dm_ref[1:2, :])

        m = jnp.maximum(y + b_ref[...], 0.0)
        m = m.astype(bf16)
        m = jnp.maximum(m, _roll_lanes(m, hw))
        m = jnp.concatenate(
            [m[:, (2 * od) * hw:(2 * od + 1) * hw]
             for od in range(m.shape[-1] // (2 * hw))], axis=1)
        m = jnp.maximum(m, _roll_lanes(m, w))

        mrows.append(m)

    mall = jnp.concatenate(mrows, axis=0)
    p2 = jnp.dot(mall, sc_ref[...],
                 preferred_element_type=jnp.float32)
    nsp = p2.shape[-1] // 2
    pall = jnp.maximum(p2[:, :nsp], p2[:, nsp:])

    psum = None
    psq = None
    for e in range(g):
        pc = pall[e * cout:(e + 1) * cout]
        pooled_ref[e] = pc.astype(bf16)
        s = jnp.sum(pc, axis=1, keepdims=True)
        sq = jnp.sum(pc * pc, axis=1, keepdims=True)
        psum = s if psum is None else psum + s
        psq = sq if psq is None else psq + sq

    psum_ref[...] = jnp.broadcast_to(psum, psum_ref.shape)
    psq_ref[...] = jnp.broadcast_to(psq, psq_ref.shape)


def _bn_kernel(pooled_ref, psum_ref, psq_ref, gamma_ref, beta_ref, o_ref,
               *, inv_count, eps):
    inv_rep = 1.0 / float(_LANES)
    s = jnp.sum(psum_ref[...], axis=1, keepdims=True) * inv_rep
    sq = jnp.sum(psq_ref[...], axis=1, keepdims=True) * inv_rep
    mean = s * inv_count
    var = jnp.maximum(sq * inv_count - mean * mean, 0.0)
    scale = jax.lax.rsqrt(var + eps) * gamma_ref[...]
    shift = beta_ref[...] - mean * scale
    o_ref[...] = pooled_ref[...].astype(jnp.float32) * scale + shift


def kernel(x, conv_w, conv_b, gamma, beta):
    eps = 1e-5
    B, Cin, D, H, W = x.shape
    Cout = conv_w.shape[0]
    Do, Ho, Wo = D // 2, H // 2, W // 2
    Nsp = Do * Ho * Wo
    S = D * H * W
    f32 = jnp.float32
    bf16 = jnp.bfloat16

    G = next(g for g in (24, 12, 8, 6, 4, 2, 1) if B % g == 0)
    NT = B // G

    x_flat = x.reshape(B, Cin, S)

    w1 = (conv_w.astype(f32).transpose(2, 0, 3, 4, 1)
          .reshape(3 * Cout, 9 * Cin).astype(bf16))
    bias = conv_b.astype(f32).reshape(Cout, 1)

    li = np.arange(S)
    hh = (li // W) % H
    ww = li % W
    dd = li // (H * W)
    hwm = np.ones((9, S), np.float32)
    for kh in range(3):
        for kw in range(3):
            bad = np.zeros(S, bool)
            if kh == 0:
                bad |= hh == 0
            if kh == 2:
                bad |= hh == H - 1
            if kw == 0:
                bad |= ww == 0
            if kw == 2:
                bad |= ww == W - 1
            hwm[kh * 3 + kw, bad] = 0.0
    dm = np.ones((2, S), np.float32)
    dm[0, dd == 0] = 0.0
    dm[1, dd == D - 1] = 0.0

    sel = np.zeros((S // 2, 2 * Nsp), np.float32)
    for od in range(Do):
        for oh in range(Ho):
            for ow in range(Wo):
                k = (od * H + 2 * oh) * W + 2 * ow
                q = (od * Ho + oh) * Wo + ow
                sel[k, q] = 1.0
                sel[k + 1, Nsp + q] = 1.0

    hwm_j = jnp.asarray(hwm, bf16)
    dm_j = jnp.asarray(dm, f32)
    sc_j = jnp.asarray(sel, bf16)

    conv_body = functools.partial(_conv_pool_kernel,
                                  dims=(G, Cin, Cout, H, W))
    pooled, psum, psq = pl.pallas_call(
        conv_body,
        out_shape=(
            jax.ShapeDtypeStruct((B, Cout, Nsp), bf16),
            jax.ShapeDtypeStruct((Cout, NT * _LANES), f32),
            jax.ShapeDtypeStruct((Cout, NT * _LANES), f32),
        ),
        grid=(NT,),
        in_specs=[
            pl.BlockSpec((G, Cin, S), lambda i: (i, 0, 0)),
            pl.BlockSpec((3 * Cout, 9 * Cin), lambda i: (0, 0)),
            pl.BlockSpec((Cout, 1), lambda i: (0, 0)),
            pl.BlockSpec((9, S), lambda i: (0, 0)),
            pl.BlockSpec((2, S), lambda i: (0, 0)),
            pl.BlockSpec((S // 2, 2 * Nsp), lambda i: (0, 0)),
        ],
        out_specs=(
            pl.BlockSpec((G, Cout, Nsp), lambda i: (i, 0, 0)),
            pl.BlockSpec((Cout, _LANES), lambda i: (0, i)),
            pl.BlockSpec((Cout, _LANES), lambda i: (0, i)),
        ),
        compiler_params=pltpu.CompilerParams(
            dimension_semantics=("parallel",)),
    )(x_flat, w1, bias, hwm_j, dm_j, sc_j)

    bn_body = functools.partial(_bn_kernel,
                                inv_count=1.0 / float(B * Nsp), eps=float(eps))
    out_flat = pl.pallas_call(
        bn_body,
        out_shape=jax.ShapeDtypeStruct((B, Cout, Nsp), f32),
        grid=(NT,),
        in_specs=[
            pl.BlockSpec((G, Cout, Nsp), lambda i: (i, 0, 0)),
            pl.BlockSpec((Cout, NT * _LANES), lambda i: (0, 0)),
            pl.BlockSpec((Cout, NT * _LANES), lambda i: (0, 0)),
            pl.BlockSpec((Cout, 1), lambda i: (0, 0)),
            pl.BlockSpec((Cout, 1), lambda i: (0, 0)),
        ],
        out_specs=pl.BlockSpec((G, Cout, Nsp), lambda i: (i, 0, 0)),
        compiler_params=pltpu.CompilerParams(
            dimension_semantics=("parallel",)),
    )(pooled, psum, psq,
      gamma.astype(f32).reshape(Cout, 1), beta.astype(f32).reshape(Cout, 1))

    return out_flat.reshape(B, Cout, Do, Ho, Wo)
